# baseline (speedup 1.0000x reference)
import jax
import jax.numpy as jnp
from jax.experimental import pallas as pl

_EPS = 1e-12


def _pass_body(bank_ref, ir_ref, qr_ref, packed_ref, msum_ref):
    nb, K2, D2 = bank_ref.shape
    D = D2 // 2
    K = 2 * K2
    itm_all = ir_ref[:, 0, :]
    qry_all = qr_ref[:, 0, :]

    inv_i = 1.0 / jnp.maximum(jnp.sqrt(jnp.sum(itm_all * itm_all, axis=1, keepdims=True)), _EPS)
    inv_q = 1.0 / jnp.maximum(jnp.sqrt(jnp.sum(qry_all * qry_all, axis=1, keepdims=True)), _EPS)

    z = jnp.zeros_like(itm_all)
    V = jnp.concatenate([
        jnp.concatenate([itm_all, z], axis=1),
        jnp.concatenate([z, itm_all], axis=1),
        jnp.concatenate([qry_all, z], axis=1),
        jnp.concatenate([z, qry_all], axis=1),
    ], axis=0)
    lane2 = jax.lax.broadcasted_iota(jnp.int32, (8, D2), 1)
    sub2 = jax.lax.broadcasted_iota(jnp.int32, (8, D2), 0)
    ones2 = jnp.where(jnp.logical_and(sub2 == 0, lane2 < D), 1.0, 0.0) \
        + jnp.where(jnp.logical_and(sub2 == 1, lane2 >= D), 1.0, 0.0)

    die, dio, dqe, dqo, nse, nso = [], [], [], [], [], []
    for s in range(nb):
        bank2 = bank_ref[s]
        MX = jax.lax.dot_general(V, bank2, (((1,), (1,)), ((), ())),
                                 preferred_element_type=jnp.float32)
        NO = jax.lax.dot_general(ones2, bank2 * bank2, (((1,), (1,)), ((), ())),
                                 preferred_element_type=jnp.float32)
        die.append(MX[s:s + 1, :])
        dio.append(MX[nb + s:nb + s + 1, :])
        dqe.append(MX[2 * nb + s:2 * nb + s + 1, :])
        dqo.append(MX[3 * nb + s:3 * nb + s + 1, :])
        nse.append(NO[0:1, :])
        nso.append(NO[1:2, :])

    d_i_e = jnp.concatenate(die, axis=0)
    d_i_o = jnp.concatenate(dio, axis=0)
    d_q_e = jnp.concatenate(dqe, axis=0)
    d_q_o = jnp.concatenate(dqo, axis=0)
    nsq_e = jnp.concatenate(nse, axis=0)
    nsq_o = jnp.concatenate(nso, axis=0)

    inv_be = 1.0 / jnp.maximum(jnp.sqrt(nsq_e), _EPS)
    inv_bo = 1.0 / jnp.maximum(jnp.sqrt(nsq_o), _EPS)
    s_i_e = d_i_e * inv_be * inv_i
    s_i_o = d_i_o * inv_bo * inv_i
    s_q_e = d_q_e * inv_be * inv_q
    s_q_o = d_q_o * inv_bo * inv_q

    e_e = jnp.exp(s_q_e)
    e_o = jnp.exp(s_q_o)
    S = (jnp.sum(e_e, axis=1, keepdims=True)
         + jnp.sum(e_o, axis=1, keepdims=True))

    m = jnp.maximum(jnp.max(s_i_e, axis=1, keepdims=True),
                    jnp.max(s_i_o, axis=1, keepdims=True))
    jio = jax.lax.broadcasted_iota(jnp.int32, (nb, K2), 1)
    cand_e = jnp.where(s_i_e >= m, 2 * jio, K)
    cand_o = jnp.where(s_i_o >= m, 2 * jio + 1, K)
    idx = jnp.minimum(jnp.min(cand_e, axis=1, keepdims=True),
                      jnp.min(cand_o, axis=1, keepdims=True))
    oh_e = (2 * jio == idx).astype(jnp.float32)
    oh_o = (2 * jio + 1 == idx).astype(jnp.float32)
    sq_best = jnp.sum(oh_e * s_q_e + oh_o * s_q_o, axis=1, keepdims=True)
    sq_0 = s_q_e[:, 0:1]

    EO = jnp.concatenate([e_e, e_o, oh_e, oh_o], axis=0)
    r_rows, rb_rows, r0_rows = [], [], []
    for s in range(nb):
        bank2 = bank_ref[s]
        R4 = jax.lax.dot_general(EO, bank2, (((1,), (0,)), ((), ())),
                                 preferred_element_type=jnp.float32)
        r_rows.append(R4[s:s + 1, 0:D] + R4[nb + s:nb + s + 1, D:2 * D])
        rb_rows.append(R4[2 * nb + s:2 * nb + s + 1, 0:D]
                       + R4[3 * nb + s:3 * nb + s + 1, D:2 * D])
        r0_rows.append(bank2[0:1, 0:D])

    R = jnp.concatenate(r_rows, axis=0)
    row_best = jnp.concatenate(rb_rows, axis=0)
    row0 = jnp.concatenate(r0_rows, axis=0)

    e_best = jnp.exp(sq_best)
    e_0 = jnp.exp(sq_0)
    new_c = 0.5 * row_best + 0.5 * itm_all
    inv_nc = 1.0 / jnp.maximum(jnp.sqrt(jnp.sum(new_c * new_c, axis=1, keepdims=True)), _EPS)
    e_new_c = jnp.exp(jnp.sum(new_c * qry_all, axis=1, keepdims=True) * inv_nc * inv_q)
    e_new_o = jnp.exp(jnp.sum(itm_all * qry_all, axis=1, keepdims=True) * inv_i * inv_q)

    A_c = e_new_c * new_c - e_best * row_best
    A_o = e_new_o * itm_all - e_0 * row0
    dS_c = e_new_c - e_best
    dS_o = e_new_o - e_0

    dlane = jax.lax.broadcasted_iota(jnp.int32, (nb, D), 1)
    scal = (jnp.where(dlane == 0, S, 0.0)
            + jnp.where(dlane == 1, dS_c, 0.0)
            + jnp.where(dlane == 2, dS_o, 0.0)
            + jnp.where(dlane == 3, m, 0.0))

    packed_ref[:, 0, :] = R
    packed_ref[:, 1, :] = A_c
    packed_ref[:, 2, :] = A_o
    packed_ref[:, 3, :] = scal

    b = pl.program_id(0)

    @pl.when(b == 0)
    def _init():
        msum_ref[...] = jnp.zeros_like(msum_ref)

    msum_ref[...] += jnp.sum(m)


def _finalize_body(packed_ref, msum_ref, out_ref):
    pk = packed_ref[...]
    B = pk.shape[0]
    R = pk[:, 0, :]
    A_c = pk[:, 1, :]
    A_o = pk[:, 2, :]
    S = pk[:, 3, 0:1]
    dS_c = pk[:, 3, 1:2]
    dS_o = pk[:, 3, 2:3]
    cond = (msum_ref[0, 0] * (1.0 / B)) >= 0.5
    S_fin = S + jnp.where(cond, dS_c, dS_o)
    R_fin = R + jnp.where(cond, A_c, A_o)
    out_ref[...] = R_fin / S_fin


def kernel(query, item, memory_bank):
    B, K, D = memory_bank.shape
    q3 = query.reshape(B, 1, D)
    i3 = item.reshape(B, 1, D)
    bank2 = memory_bank.reshape(B, K // 2, 2 * D)

    BPB = 8
    packed, msum = pl.pallas_call(
        _pass_body,
        grid=(B // BPB,),
        in_specs=[
            pl.BlockSpec((BPB, K // 2, 2 * D), lambda b: (b, 0, 0)),
            pl.BlockSpec((BPB, 1, D), lambda b: (b, 0, 0)),
            pl.BlockSpec((BPB, 1, D), lambda b: (b, 0, 0)),
        ],
        out_specs=[
            pl.BlockSpec((BPB, 4, D), lambda b: (b, 0, 0)),
            pl.BlockSpec((1, 128), lambda b: (0, 0)),
        ],
        out_shape=[
            jax.ShapeDtypeStruct((B, 4, D), jnp.float32),
            jax.ShapeDtypeStruct((1, 128), jnp.float32),
        ],
    )(bank2, i3, q3)

    retrieved = pl.pallas_call(
        _finalize_body,
        out_shape=jax.ShapeDtypeStruct((B, D), jnp.float32),
    )(packed, msum)
    return retrieved
